# baseline (speedup 1.0000x reference)
import jax
import jax.numpy as jnp
from jax.experimental import pallas as pl
from jax.experimental.pallas import tpu as pltpu

N = 100000
CH = 128
B = 64
BLK = 4000
NB = N // BLK
NEG = -1e30


def _body(x_ref, st_ref, en_ref, u_ref, qw1_ref, qb1_ref, qw2_ref, qb2_ref,
          nw1_ref, nb1_ref, nw2_ref, nb2_ref, gw1_ref, gb1_ref, gw2_ref,
          out_ref, q_s, m_s, s_s, acc_s):
    i = pl.program_id(0)

    @pl.when(i == 0)
    def _init():
        uq = jnp.maximum(
            jnp.dot(u_ref[:], qw1_ref[:], preferred_element_type=jnp.float32)
            + qb1_ref[:], 0.0)
        q_s[0:B, :] = jnp.dot(uq, qw2_ref[:],
                              preferred_element_type=jnp.float32) + qb2_ref[:]
        q_s[B:CH, :] = jnp.zeros((CH - B, CH), jnp.float32)
        m_s[:] = jnp.full((1, CH), NEG, jnp.float32)
        s_s[:] = jnp.zeros((1, CH), jnp.float32)
        acc_s[:] = jnp.zeros((CH, CH), jnp.float32)

    def half(xs, base):
        h1 = jnp.maximum(
            jnp.dot(xs, nw1_ref[:], preferred_element_type=jnp.float32)
            + nb1_ref[:], 0.0)
        h = jnp.dot(h1, nw2_ref[:], preferred_element_type=jnp.float32) \
            + nb2_ref[:]
        row = jax.lax.broadcasted_iota(jnp.int32, (BLK // 2, CH), 0) + base
        oh = (row >= st_ref[:]) & (row < en_ref[:])
        qg = jnp.dot(oh.astype(jnp.float32), q_s[:],
                     preferred_element_type=jnp.float32)
        gin = qg * h
        g1 = jnp.maximum(
            jnp.dot(gin, gw1_ref[:], preferred_element_type=jnp.float32)
            + gb1_ref[:], 0.0)
        gate = jnp.sum(g1 * gw2_ref[:].reshape(1, CH), axis=1, keepdims=True)
        bm = jnp.max(jnp.where(oh, gate, NEG), axis=0, keepdims=True)
        return h, oh, gate, bm

    ha, oha, gatea, bma = half(x_ref[0:BLK // 2], i * BLK)
    hb, ohb, gateb, bmb = half(x_ref[BLK // 2:BLK], i * BLK + BLK // 2)

    m_old = m_s[:]
    m_new = jnp.maximum(m_old, jnp.maximum(bma, bmb))
    scale = jnp.exp(m_old - m_new)
    ohea = jnp.where(oha, jnp.exp(gatea - m_new), 0.0)
    oheb = jnp.where(ohb, jnp.exp(gateb - m_new), 0.0)

    s_s[:] = s_s[:] * scale \
        + jnp.sum(ohea, axis=0, keepdims=True) \
        + jnp.sum(oheb, axis=0, keepdims=True)
    acc_s[:] = acc_s[:] * scale + jax.lax.dot_general(
        ha, ohea, (((0,), (0,)), ((), ())),
        preferred_element_type=jnp.float32) + jax.lax.dot_general(
        hb, oheb, (((0,), (0,)), ((), ())),
        preferred_element_type=jnp.float32)
    m_s[:] = m_new

    @pl.when(i == NB - 1)
    def _fin():
        out_ref[:] = jnp.transpose(acc_s[:] / (s_s[:] + 1e-16))[0:B, :]


def kernel(x, u, batch, size, gate_w1, gate_b1, gate_w2, gate_b2,
           node_w1, node_b1, node_w2, node_b2,
           ques_w1, ques_b1, ques_w2, ques_b2):
    num_seg = u.shape[0]
    off = jnp.asarray(size, jnp.int32) - jnp.int32(num_seg)
    q = jnp.arange(CH + 1, dtype=jnp.int32) - off
    bounds = jnp.sum((batch.astype(jnp.int32)[:, None] < q[None, :])
                     .astype(jnp.int32), axis=0)
    starts = bounds[:CH].reshape(1, CH)
    ends = bounds[1:CH + 1].reshape(1, CH)

    full = pl.BlockSpec((CH, CH), lambda i: (0, 0))
    row = pl.BlockSpec((1, CH), lambda i: (0, 0))
    out = pl.pallas_call(
        _body,
        grid=(NB,),
        in_specs=[
            pl.BlockSpec((BLK, CH), lambda i: (i, 0)),
            row, row,
            pl.BlockSpec((B, CH), lambda i: (0, 0)),
            full, row,
            full, row,
            full, row,
            full, row,
            full, row,
            pl.BlockSpec((CH, 1), lambda i: (0, 0)),
        ],
        out_specs=pl.BlockSpec((B, CH), lambda i: (0, 0)),
        out_shape=jax.ShapeDtypeStruct((B, CH), jnp.float32),
        scratch_shapes=[
            pltpu.VMEM((CH, CH), jnp.float32),
            pltpu.VMEM((1, CH), jnp.float32),
            pltpu.VMEM((1, CH), jnp.float32),
            pltpu.VMEM((CH, CH), jnp.float32),
        ],
    )(x, starts, ends, u,
      ques_w1, ques_b1.reshape(1, CH), ques_w2, ques_b2.reshape(1, CH),
      node_w1, node_b1.reshape(1, CH), node_w2, node_b2.reshape(1, CH),
      gate_w1, gate_b1.reshape(1, CH), gate_w2)
    return out


# speedup vs baseline: 1.0050x; 1.0050x over previous
import jax
import jax.numpy as jnp
from jax.experimental import pallas as pl
from jax.experimental.pallas import tpu as pltpu

N = 100000
CH = 128
B = 64
BLK = 4000
NB = N // BLK
NEG = -1e30


def _body(x_ref, st_ref, en_ref, u_ref, qw1_ref, qb1_ref, qw2_ref, qb2_ref,
          nw1_ref, nb1_ref, nw2_ref, nb2_ref, gw1_ref, gb1_ref, gw2_ref,
          out_ref, q_s, m_s, s_s, acc_s):
    i = pl.program_id(0)

    @pl.when(i == 0)
    def _init():
        uq = jnp.maximum(
            jnp.dot(u_ref[:], qw1_ref[:], preferred_element_type=jnp.float32)
            + qb1_ref[:], 0.0)
        q_s[0:B, :] = jnp.dot(uq, qw2_ref[:],
                              preferred_element_type=jnp.float32) + qb2_ref[:]
        q_s[B:CH, :] = jnp.zeros((CH - B, CH), jnp.float32)
        m_s[:] = jnp.full((1, CH), NEG, jnp.float32)
        s_s[:] = jnp.zeros((1, CH), jnp.float32)
        acc_s[:] = jnp.zeros((CH, CH), jnp.float32)

    x = x_ref[:]
    h1 = jnp.maximum(
        jnp.dot(x, nw1_ref[:], preferred_element_type=jnp.float32)
        + nb1_ref[:], 0.0)
    h = jnp.dot(h1, nw2_ref[:], preferred_element_type=jnp.float32) \
        + nb2_ref[:]

    row = jax.lax.broadcasted_iota(jnp.int32, (BLK, CH), 0) + i * BLK
    oh = (row >= st_ref[:]) & (row < en_ref[:])
    ohf = oh.astype(jnp.float32)

    qg = jnp.dot(ohf, q_s[:], preferred_element_type=jnp.float32)
    gin = qg * h
    g1 = jnp.maximum(
        jnp.dot(gin, gw1_ref[:], preferred_element_type=jnp.float32)
        + gb1_ref[:], 0.0)
    gate = jnp.sum(g1 * gw2_ref[:].reshape(1, CH), axis=1, keepdims=True)

    bm = jnp.max(jnp.where(oh, gate, NEG), axis=0, keepdims=True)
    m_old = m_s[:]
    m_new = jnp.maximum(m_old, bm)
    scale = jnp.exp(m_old - m_new)
    ohe = jnp.where(oh, jnp.exp(gate - m_new), 0.0)

    s_s[:] = s_s[:] * scale + jnp.sum(ohe, axis=0, keepdims=True)
    acc_s[:] = acc_s[:] * scale + jax.lax.dot_general(
        h, ohe, (((0,), (0,)), ((), ())),
        preferred_element_type=jnp.float32)
    m_s[:] = m_new

    @pl.when(i == NB - 1)
    def _fin():
        out_ref[:] = jnp.transpose(acc_s[:] / (s_s[:] + 1e-16))[0:B, :]


def kernel(x, u, batch, size, gate_w1, gate_b1, gate_w2, gate_b2,
           node_w1, node_b1, node_w2, node_b2,
           ques_w1, ques_b1, ques_w2, ques_b2):
    num_seg = u.shape[0]
    off = jnp.asarray(size, jnp.int32) - jnp.int32(num_seg)
    q = jnp.arange(CH + 1, dtype=jnp.int32) - off
    bounds = jnp.sum((batch.astype(jnp.int32)[:, None] < q[None, :])
                     .astype(jnp.int32), axis=0)
    starts = bounds[:CH].reshape(1, CH)
    ends = bounds[1:CH + 1].reshape(1, CH)

    full = pl.BlockSpec((CH, CH), lambda i: (0, 0))
    row = pl.BlockSpec((1, CH), lambda i: (0, 0))
    out = pl.pallas_call(
        _body,
        grid=(NB,),
        in_specs=[
            pl.BlockSpec((BLK, CH), lambda i: (i, 0)),
            row, row,
            pl.BlockSpec((B, CH), lambda i: (0, 0)),
            full, row,
            full, row,
            full, row,
            full, row,
            full, row,
            pl.BlockSpec((CH, 1), lambda i: (0, 0)),
        ],
        out_specs=pl.BlockSpec((B, CH), lambda i: (0, 0)),
        out_shape=jax.ShapeDtypeStruct((B, CH), jnp.float32),
        scratch_shapes=[
            pltpu.VMEM((CH, CH), jnp.float32),
            pltpu.VMEM((1, CH), jnp.float32),
            pltpu.VMEM((1, CH), jnp.float32),
            pltpu.VMEM((CH, CH), jnp.float32),
        ],
    )(x, starts, ends, u,
      ques_w1, ques_b1.reshape(1, CH), ques_w2, ques_b2.reshape(1, CH),
      node_w1, node_b1.reshape(1, CH), node_w2, node_b2.reshape(1, CH),
      gate_w1, gate_b1.reshape(1, CH), gate_w2)
    return out
